# baseline (speedup 1.0000x reference)
import functools

import jax
import jax.numpy as jnp
from jax import lax
from jax.experimental import pallas as pl
from jax.experimental.pallas import tpu as pltpu
from jax.experimental.pallas import tpu_sc as plsc

N = 10000
E = 320000
D = 128
R = 8
NB = 4
R1 = R + 1

NC = 2
NS = 16
NW = NC * NS
EPW = E // NW
C = 80
NCHUNK = EPW // C
RSTRIDE = 624
RSIZE = 640

BN = 1000
NBLK = N // BN


GROWS = NBLK * 8
GCOLS = E // GROWS


def _transform(xs, coeff, V, W_loop, idx2=None):
    n_in = len(xs)
    with_gidx = idx2 is not None

    def body(*refs):
        coeff_ref = refs[0]
        v_ref = refs[1]
        wl_ref = refs[2]
        x_refs = refs[3:3 + n_in]
        rest = refs[3 + n_in:]
        if with_gidx:
            s_ref, e_ref, o_ref, g_ref, w_scr = rest
        else:
            (o_ref, w_scr) = rest[:2]
        nb = pl.program_id(0)
        r = pl.program_id(1)

        @pl.when(jnp.logical_and(nb == 0, r < R))
        def _():
            w = coeff_ref[r, 0] * v_ref[0]
            for b in range(1, NB):
                w = w + coeff_ref[r, b] * v_ref[b]
            w_scr[r] = w

        @pl.when(jnp.logical_and(nb == 0, r == R))
        def _():
            w_scr[R] = wl_ref[...]

        if with_gidx:
            g_ref[...] = e_ref[...] * N + s_ref[...]

        x = x_refs[0][...]
        for xr in x_refs[1:]:
            x = x + xr[...]
        o_ref[...] = jnp.dot(x, w_scr[r], preferred_element_type=jnp.float32)

    in_specs = [
        pl.BlockSpec(memory_space=pltpu.SMEM),
        pl.BlockSpec((NB, D, D), lambda nb, r: (0, 0, 0)),
        pl.BlockSpec((D, D), lambda nb, r: (0, 0)),
    ]
    in_specs += [
        pl.BlockSpec((BN, D), functools.partial(lambda off, nb, r: (off + nb, 0), off))
        for (_, off) in xs
    ]
    args = [coeff, V, W_loop] + [a for (a, _) in xs]
    out_shape = jax.ShapeDtypeStruct((R1 * N, D), jnp.float32)
    out_specs = pl.BlockSpec((BN, D), lambda nb, r: (r * NBLK + nb, 0))
    if with_gidx:
        in_specs += [
            pl.BlockSpec((8, GCOLS), lambda nb, r: (nb, 0)),
            pl.BlockSpec((8, GCOLS), lambda nb, r: (nb, 0)),
        ]
        args += [idx2[0], idx2[1]]
        out_shape = (out_shape,
                     jax.ShapeDtypeStruct((GROWS, GCOLS), jnp.int32))
        out_specs = (out_specs, pl.BlockSpec((8, GCOLS), lambda nb, r: (nb, 0)))

    return pl.pallas_call(
        body,
        grid=(NBLK, R1),
        out_shape=out_shape,
        in_specs=in_specs,
        out_specs=out_specs,
        scratch_shapes=[pltpu.VMEM((R1, D, D), jnp.float32)],
    )(*args)


def _sc_message(Y, gidx, dst3):
    mesh = plsc.VectorSubcoreMesh(
        core_axis_name="c", subcore_axis_name="s",
        num_cores=NC, num_subcores=NS)

    @functools.partial(
        pl.kernel,
        out_type=jax.ShapeDtypeStruct((NC * N, D), jnp.float32),
        mesh=mesh,
        scratch_types=[
            pltpu.VMEM((NCHUNK, C), jnp.int32),
            pltpu.VMEM((EPW,), jnp.int32),
            pltpu.VMEM((C, D), jnp.float32),
            pltpu.VMEM((C, D), jnp.float32),
            pltpu.VMEM_SHARED((N, D), jnp.float32),
            pltpu.SemaphoreType.DMA,
            pltpu.SemaphoreType.DMA,
        ],
    )
    def k(y_hbm, gidx_hbm, dst_hbm, out_hbm,
          dstm, gidxv, rows0, rows1, agg, gs0, gs1):
        c = lax.axis_index("c")
        s = lax.axis_index("s")
        wid = c * NS + s
        row0 = s * RSTRIDE
        ebase = wid * EPW

        cp_gi = pltpu.async_copy(gidx_hbm.at[pl.ds(ebase, EPW)], gidxv, gs0)
        cp_dm = pltpu.async_copy(dst_hbm.at[wid], dstm, gs1)
        def zr(i, carry):
            for jj in range(D // 16):
                rows0[i, pl.ds(jj * 16, 16)] = jnp.zeros((16,), jnp.float32)
            return carry
        lax.fori_loop(0, C, zr, 0)
        for t in range(RSIZE // C):
            pltpu.sync_copy(rows0, agg.at[pl.ds(row0 + t * C, C)])
        cp_gi.wait()
        cp_dm.wait()

        plsc.subcore_barrier()

        wg0 = pltpu.make_async_copy(y_hbm.at[gidxv.at[pl.ds(0, C)]], rows0, gs0)
        wg1 = pltpu.make_async_copy(y_hbm.at[gidxv.at[pl.ds(0, C)]], rows1, gs1)
        pltpu.async_copy(y_hbm.at[gidxv.at[pl.ds(0, C)]], rows0, gs0)
        pltpu.async_copy(y_hbm.at[gidxv.at[pl.ds(C, C)]], rows1, gs1)

        def pair(jj, carry):
            j0 = 2 * jj
            wg0.wait()
            pltpu.sync_copy(rows0, agg.at[dstm.at[j0]], add=True)
            pltpu.async_copy(
                y_hbm.at[gidxv.at[pl.ds((j0 + 2) * C, C)]], rows0, gs0)
            wg1.wait()
            pltpu.sync_copy(rows1, agg.at[dstm.at[j0 + 1]], add=True)

            @pl.when(jj < (NCHUNK - 3) // 2)
            def _():
                pltpu.async_copy(
                    y_hbm.at[gidxv.at[pl.ds((j0 + 3) * C, C)]], rows1, gs1)
            return carry

        lax.fori_loop(0, (NCHUNK - 1) // 2, pair, 0)
        wg0.wait()
        pltpu.sync_copy(rows0, agg.at[dstm.at[NCHUNK - 1]], add=True)

        plsc.subcore_barrier()
        pltpu.sync_copy(agg.at[pl.ds(row0, RSIZE)],
                        out_hbm.at[pl.ds(c * N + row0, RSIZE)])

    return k(Y, gidx, dst3)


def _final_add(P, Y):
    def body(p0_ref, p1_ref, l_ref, o_ref):
        o_ref[...] = p0_ref[...] + p1_ref[...] + l_ref[...]

    return pl.pallas_call(
        body,
        grid=(NBLK,),
        out_shape=jax.ShapeDtypeStruct((N, D), jnp.float32),
        in_specs=[
            pl.BlockSpec((BN, D), lambda i: (i, 0)),
            pl.BlockSpec((BN, D), lambda i: (NBLK + i, 0)),
            pl.BlockSpec((BN, D), lambda i: (R * NBLK + i, 0)),
        ],
        out_specs=pl.BlockSpec((BN, D), lambda i: (i, 0)),
    )(P, P, Y)


@jax.jit
def kernel(emb, edge_index, etype, V, coeff, W_loop):
    src2 = edge_index[0].reshape(GROWS, GCOLS)
    et2 = etype.reshape(GROWS, GCOLS)
    dst3 = edge_index[1].reshape(NW, NCHUNK, C)

    Y1, gidx2 = _transform([(emb, 0)], coeff, V, W_loop,
                           idx2=(src2, et2))
    gidx = gidx2.reshape(E)
    P1 = _sc_message(Y1, gidx, dst3)
    Y2 = _transform(
        [(P1, 0), (P1, NBLK), (Y1, R * NBLK), (emb, 0)], coeff, V, W_loop)
    P2 = _sc_message(Y2, gidx, dst3)
    return _final_add(P2, Y2)


# speedup vs baseline: 1.2317x; 1.2317x over previous
import functools

import jax
import jax.numpy as jnp
from jax import lax
from jax.experimental import pallas as pl
from jax.experimental.pallas import tpu as pltpu
from jax.experimental.pallas import tpu_sc as plsc

N = 10000
E = 320000
D = 128
R = 8
NB = 4
R1 = R + 1

NC = 2
NS = 16
NW = NC * NS
EPW = E // NW
C = 80
NCHUNK = EPW // C
RSTRIDE = 624
RSIZE = 640

BN = 1000
NBLK = N // BN


GROWS = NBLK * 8
GCOLS = E // GROWS


def _transform(xs, coeff, V, W_loop, idx2=None):
    n_in = len(xs)
    with_gidx = idx2 is not None

    def body(*refs):
        coeff_ref = refs[0]
        v_ref = refs[1]
        wl_ref = refs[2]
        x_refs = refs[3:3 + n_in]
        rest = refs[3 + n_in:]
        if with_gidx:
            s_ref, e_ref, o_ref, g_ref, w_scr = rest
        else:
            (o_ref, w_scr) = rest[:2]
        nb = pl.program_id(0)

        @pl.when(nb == 0)
        def _():
            for r in range(R):
                w = coeff_ref[r, 0] * v_ref[0]
                for b in range(1, NB):
                    w = w + coeff_ref[r, b] * v_ref[b]
                w_scr[r] = w
            w_scr[R] = wl_ref[...]

        if with_gidx:
            g_ref[...] = e_ref[...] * N + s_ref[...]

        x = x_refs[0][...]
        for xr in x_refs[1:]:
            x = x + xr[...]
        for r in range(R1):
            o_ref[r] = jnp.dot(x, w_scr[r], preferred_element_type=jnp.float32)

    in_specs = [
        pl.BlockSpec(memory_space=pltpu.SMEM),
        pl.BlockSpec((NB, D, D), lambda nb: (0, 0, 0)),
        pl.BlockSpec((D, D), lambda nb: (0, 0)),
    ]
    in_specs += [
        pl.BlockSpec((BN, D), functools.partial(lambda off, nb: (off + nb, 0), off))
        for (_, off) in xs
    ]
    args = [coeff, V, W_loop] + [a for (a, _) in xs]
    out_shape = jax.ShapeDtypeStruct((R1, N, D), jnp.float32)
    out_specs = pl.BlockSpec((R1, BN, D), lambda nb: (0, nb, 0))
    if with_gidx:
        in_specs += [
            pl.BlockSpec((8, GCOLS), lambda nb: (nb, 0)),
            pl.BlockSpec((8, GCOLS), lambda nb: (nb, 0)),
        ]
        args += [idx2[0], idx2[1]]
        out_shape = (out_shape,
                     jax.ShapeDtypeStruct((GROWS, GCOLS), jnp.int32))
        out_specs = (out_specs, pl.BlockSpec((8, GCOLS), lambda nb: (nb, 0)))

    out = pl.pallas_call(
        body,
        grid=(NBLK,),
        out_shape=out_shape,
        in_specs=in_specs,
        out_specs=out_specs,
        scratch_shapes=[pltpu.VMEM((R1, D, D), jnp.float32)],
    )(*args)
    if with_gidx:
        return out[0].reshape(R1 * N, D), out[1]
    return out.reshape(R1 * N, D)


def _sc_message(Y, gidx, dst3):
    mesh = plsc.VectorSubcoreMesh(
        core_axis_name="c", subcore_axis_name="s",
        num_cores=NC, num_subcores=NS)

    @functools.partial(
        pl.kernel,
        out_type=jax.ShapeDtypeStruct((NC * N, D), jnp.float32),
        mesh=mesh,
        scratch_types=[
            pltpu.VMEM((NCHUNK, C), jnp.int32),
            pltpu.VMEM((EPW,), jnp.int32),
            pltpu.VMEM((C, D), jnp.float32),
            pltpu.VMEM((C, D), jnp.float32),
            pltpu.VMEM_SHARED((N, D), jnp.float32),
            pltpu.SemaphoreType.DMA,
            pltpu.SemaphoreType.DMA,
        ],
    )
    def k(y_hbm, gidx_hbm, dst_hbm, out_hbm,
          dstm, gidxv, rows0, rows1, agg, gs0, gs1):
        c = lax.axis_index("c")
        s = lax.axis_index("s")
        wid = c * NS + s
        row0 = s * RSTRIDE
        ebase = wid * EPW

        cp_gi = pltpu.async_copy(gidx_hbm.at[pl.ds(ebase, EPW)], gidxv, gs0)
        cp_dm = pltpu.async_copy(dst_hbm.at[wid], dstm, gs1)
        def zr(i, carry):
            for jj in range(D // 16):
                rows0[i, pl.ds(jj * 16, 16)] = jnp.zeros((16,), jnp.float32)
            return carry
        lax.fori_loop(0, C, zr, 0)
        for t in range(RSIZE // C):
            pltpu.sync_copy(rows0, agg.at[pl.ds(row0 + t * C, C)])
        cp_gi.wait()
        cp_dm.wait()

        plsc.subcore_barrier()

        wg0 = pltpu.make_async_copy(y_hbm.at[gidxv.at[pl.ds(0, C)]], rows0, gs0)
        wg1 = pltpu.make_async_copy(y_hbm.at[gidxv.at[pl.ds(0, C)]], rows1, gs1)
        pltpu.async_copy(y_hbm.at[gidxv.at[pl.ds(0, C)]], rows0, gs0)
        pltpu.async_copy(y_hbm.at[gidxv.at[pl.ds(C, C)]], rows1, gs1)

        def pair(jj, carry):
            j0 = 2 * jj
            wg0.wait()
            pltpu.sync_copy(rows0, agg.at[dstm.at[j0]], add=True)
            pltpu.async_copy(
                y_hbm.at[gidxv.at[pl.ds((j0 + 2) * C, C)]], rows0, gs0)
            wg1.wait()
            pltpu.sync_copy(rows1, agg.at[dstm.at[j0 + 1]], add=True)

            @pl.when(jj < (NCHUNK - 3) // 2)
            def _():
                pltpu.async_copy(
                    y_hbm.at[gidxv.at[pl.ds((j0 + 3) * C, C)]], rows1, gs1)
            return carry

        lax.fori_loop(0, (NCHUNK - 1) // 2, pair, 0)
        wg0.wait()
        pltpu.sync_copy(rows0, agg.at[dstm.at[NCHUNK - 1]], add=True)

        plsc.subcore_barrier()
        pltpu.sync_copy(agg.at[pl.ds(row0, RSIZE)],
                        out_hbm.at[pl.ds(c * N + row0, RSIZE)])

    return k(Y, gidx, dst3)


def _final_add(P, Y):
    def body(p0_ref, p1_ref, l_ref, o_ref):
        o_ref[...] = p0_ref[...] + p1_ref[...] + l_ref[...]

    return pl.pallas_call(
        body,
        grid=(NBLK,),
        out_shape=jax.ShapeDtypeStruct((N, D), jnp.float32),
        in_specs=[
            pl.BlockSpec((BN, D), lambda i: (i, 0)),
            pl.BlockSpec((BN, D), lambda i: (NBLK + i, 0)),
            pl.BlockSpec((BN, D), lambda i: (R * NBLK + i, 0)),
        ],
        out_specs=pl.BlockSpec((BN, D), lambda i: (i, 0)),
    )(P, P, Y)


@jax.jit
def kernel(emb, edge_index, etype, V, coeff, W_loop):
    src2 = edge_index[0].reshape(GROWS, GCOLS)
    et2 = etype.reshape(GROWS, GCOLS)
    dst3 = edge_index[1].reshape(NW, NCHUNK, C)

    Y1, gidx2 = _transform([(emb, 0)], coeff, V, W_loop,
                           idx2=(src2, et2))
    gidx = gidx2.reshape(E)
    P1 = _sc_message(Y1, gidx, dst3)
    Y2 = _transform(
        [(P1, 0), (P1, NBLK), (Y1, R * NBLK), (emb, 0)], coeff, V, W_loop)
    P2 = _sc_message(Y2, gidx, dst3)
    return _final_add(P2, Y2)


# speedup vs baseline: 1.2707x; 1.0316x over previous
import functools

import jax
import jax.numpy as jnp
from jax import lax
from jax.experimental import pallas as pl
from jax.experimental.pallas import tpu as pltpu
from jax.experimental.pallas import tpu_sc as plsc

N = 10000
E = 320000
D = 128
R = 8
NB = 4
R1 = R + 1

NC = 2
NS = 16
NW = NC * NS
EPW = E // NW
C = 80
NCHUNK = EPW // C
RSTRIDE = 624
RSIZE = 640

BN = 2000
NBLK = N // BN


GROWS = NBLK * 8
GCOLS = E // GROWS


def _transform(xs, coeff, V, W_loop, idx2=None):
    n_in = len(xs)
    with_gidx = idx2 is not None

    def body(*refs):
        coeff_ref = refs[0]
        v_ref = refs[1]
        wl_ref = refs[2]
        x_refs = refs[3:3 + n_in]
        rest = refs[3 + n_in:]
        if with_gidx:
            s_ref, e_ref, o_ref, g_ref, w_scr = rest
        else:
            (o_ref, w_scr) = rest[:2]
        nb = pl.program_id(0)

        @pl.when(nb == 0)
        def _():
            for r in range(R):
                w = coeff_ref[r, 0] * v_ref[0]
                for b in range(1, NB):
                    w = w + coeff_ref[r, b] * v_ref[b]
                w_scr[r] = w
            w_scr[R] = wl_ref[...]

        if with_gidx:
            g_ref[...] = e_ref[...] * N + s_ref[...]

        x = x_refs[0][...]
        for xr in x_refs[1:]:
            x = x + xr[...]
        for r in range(R1):
            o_ref[r] = jnp.dot(x, w_scr[r], preferred_element_type=jnp.float32)

    in_specs = [
        pl.BlockSpec(memory_space=pltpu.SMEM),
        pl.BlockSpec((NB, D, D), lambda nb: (0, 0, 0)),
        pl.BlockSpec((D, D), lambda nb: (0, 0)),
    ]
    in_specs += [
        pl.BlockSpec((BN, D), functools.partial(lambda off, nb: (off + nb, 0), off))
        for (_, off) in xs
    ]
    args = [coeff, V, W_loop] + [a for (a, _) in xs]
    out_shape = jax.ShapeDtypeStruct((R1, N, D), jnp.float32)
    out_specs = pl.BlockSpec((R1, BN, D), lambda nb: (0, nb, 0))
    if with_gidx:
        in_specs += [
            pl.BlockSpec((8, GCOLS), lambda nb: (nb, 0)),
            pl.BlockSpec((8, GCOLS), lambda nb: (nb, 0)),
        ]
        args += [idx2[0], idx2[1]]
        out_shape = (out_shape,
                     jax.ShapeDtypeStruct((GROWS, GCOLS), jnp.int32))
        out_specs = (out_specs, pl.BlockSpec((8, GCOLS), lambda nb: (nb, 0)))

    out = pl.pallas_call(
        body,
        grid=(NBLK,),
        out_shape=out_shape,
        in_specs=in_specs,
        out_specs=out_specs,
        scratch_shapes=[pltpu.VMEM((R1, D, D), jnp.float32)],
    )(*args)
    if with_gidx:
        return out[0].reshape(R1 * N, D), out[1]
    return out.reshape(R1 * N, D)


def _sc_message(Y, gidx, dst3):
    mesh = plsc.VectorSubcoreMesh(
        core_axis_name="c", subcore_axis_name="s",
        num_cores=NC, num_subcores=NS)

    @functools.partial(
        pl.kernel,
        out_type=jax.ShapeDtypeStruct((NC * N, D), jnp.float32),
        mesh=mesh,
        scratch_types=[
            pltpu.VMEM((NCHUNK, C), jnp.int32),
            pltpu.VMEM((EPW,), jnp.int32),
            pltpu.VMEM((C, D), jnp.float32),
            pltpu.VMEM((C, D), jnp.float32),
            pltpu.VMEM_SHARED((N, D), jnp.float32),
            pltpu.SemaphoreType.DMA,
            pltpu.SemaphoreType.DMA,
        ],
    )
    def k(y_hbm, gidx_hbm, dst_hbm, out_hbm,
          dstm, gidxv, rows0, rows1, agg, gs0, gs1):
        c = lax.axis_index("c")
        s = lax.axis_index("s")
        wid = c * NS + s
        row0 = s * RSTRIDE
        ebase = wid * EPW

        cp_gi = pltpu.async_copy(gidx_hbm.at[pl.ds(ebase, EPW)], gidxv, gs0)
        cp_dm = pltpu.async_copy(dst_hbm.at[wid], dstm, gs1)
        def zr(i, carry):
            for jj in range(D // 16):
                rows0[i, pl.ds(jj * 16, 16)] = jnp.zeros((16,), jnp.float32)
            return carry
        lax.fori_loop(0, C, zr, 0)
        for t in range(RSIZE // C):
            pltpu.sync_copy(rows0, agg.at[pl.ds(row0 + t * C, C)])
        cp_gi.wait()
        cp_dm.wait()

        plsc.subcore_barrier()

        wg0 = pltpu.make_async_copy(y_hbm.at[gidxv.at[pl.ds(0, C)]], rows0, gs0)
        wg1 = pltpu.make_async_copy(y_hbm.at[gidxv.at[pl.ds(0, C)]], rows1, gs1)
        pltpu.async_copy(y_hbm.at[gidxv.at[pl.ds(0, C)]], rows0, gs0)
        pltpu.async_copy(y_hbm.at[gidxv.at[pl.ds(C, C)]], rows1, gs1)

        def pair(jj, carry):
            j0 = 2 * jj
            wg0.wait()
            pltpu.sync_copy(rows0, agg.at[dstm.at[j0]], add=True)
            pltpu.async_copy(
                y_hbm.at[gidxv.at[pl.ds((j0 + 2) * C, C)]], rows0, gs0)
            wg1.wait()
            pltpu.sync_copy(rows1, agg.at[dstm.at[j0 + 1]], add=True)

            @pl.when(jj < (NCHUNK - 3) // 2)
            def _():
                pltpu.async_copy(
                    y_hbm.at[gidxv.at[pl.ds((j0 + 3) * C, C)]], rows1, gs1)
            return carry

        lax.fori_loop(0, (NCHUNK - 1) // 2, pair, 0)
        wg0.wait()
        pltpu.sync_copy(rows0, agg.at[dstm.at[NCHUNK - 1]], add=True)

        plsc.subcore_barrier()
        pltpu.sync_copy(agg.at[pl.ds(row0, RSIZE)],
                        out_hbm.at[pl.ds(c * N + row0, RSIZE)])

    return k(Y, gidx, dst3)


def _final_add(P, Y):
    def body(p0_ref, p1_ref, l_ref, o_ref):
        o_ref[...] = p0_ref[...] + p1_ref[...] + l_ref[...]

    return pl.pallas_call(
        body,
        grid=(NBLK,),
        out_shape=jax.ShapeDtypeStruct((N, D), jnp.float32),
        in_specs=[
            pl.BlockSpec((BN, D), lambda i: (i, 0)),
            pl.BlockSpec((BN, D), lambda i: (NBLK + i, 0)),
            pl.BlockSpec((BN, D), lambda i: (R * NBLK + i, 0)),
        ],
        out_specs=pl.BlockSpec((BN, D), lambda i: (i, 0)),
    )(P, P, Y)


@jax.jit
def kernel(emb, edge_index, etype, V, coeff, W_loop):
    src2 = edge_index[0].reshape(GROWS, GCOLS)
    et2 = etype.reshape(GROWS, GCOLS)
    dst3 = edge_index[1].reshape(NW, NCHUNK, C)

    Y1, gidx2 = _transform([(emb, 0)], coeff, V, W_loop,
                           idx2=(src2, et2))
    gidx = gidx2.reshape(E)
    P1 = _sc_message(Y1, gidx, dst3)
    Y2 = _transform(
        [(P1, 0), (P1, NBLK), (Y1, R * NBLK), (emb, 0)], coeff, V, W_loop)
    P2 = _sc_message(Y2, gidx, dst3)
    return _final_add(P2, Y2)
